# Initial kernel scaffold; baseline (speedup 1.0000x reference)
import jax
import jax.numpy as jnp
from jax.experimental import pallas as pl


def kernel(inputs, pos_protomemory, neg_protomemory, targets, indexes):
    raise NotImplementedError("write your pallas kernel here")


# speedup vs baseline: 13.0672x; 13.0672x over previous
import functools

import jax
import jax.numpy as jnp
from jax import lax
from jax.experimental import pallas as pl
from jax.experimental.pallas import tpu as pltpu
from jax.experimental.pallas import tpu_sc as plsc

B = 1024
D = 256
N = 100000
TEMP = 0.05
INV_TEMP = 1.0 / TEMP

ROWS_PER_BLOCK = 2000
NUM_BLOCKS = N // ROWS_PER_BLOCK


def _make_sc_gather():
  info = plsc.get_sparse_core_info()
  nw = info.num_cores * info.num_subcores
  b_per_w = B // nw
  mesh = plsc.VectorSubcoreMesh(core_axis_name="c", subcore_axis_name="s")

  @functools.partial(
      pl.kernel,
      mesh=mesh,
      out_type=[
          jax.ShapeDtypeStruct((B, D), jnp.float32),
          jax.ShapeDtypeStruct((B, D), jnp.float32),
      ],
      scratch_types=[
          pltpu.VMEM((b_per_w,), jnp.int32),
          pltpu.VMEM((b_per_w, D), jnp.float32),
          pltpu.VMEM((b_per_w, D), jnp.float32),
          pltpu.SemaphoreType.DMA,
          pltpu.SemaphoreType.DMA,
      ],
  )
  def sc_gather(pos_hbm, neg_hbm, tgt_hbm, out_p, out_n,
                idx_v, rows_p, rows_n, sem_p, sem_n):
    wid = lax.axis_index("s") * info.num_cores + lax.axis_index("c")
    base = wid * b_per_w
    pltpu.sync_copy(tgt_hbm.at[pl.ds(base, b_per_w)], idx_v)
    dma_p = pltpu.async_copy(pos_hbm.at[idx_v], rows_p, sem_p)
    dma_n = pltpu.async_copy(neg_hbm.at[idx_v], rows_n, sem_n)
    dma_p.wait()
    dma_n.wait()
    pltpu.sync_copy(rows_p, out_p.at[pl.ds(base, b_per_w)])
    pltpu.sync_copy(rows_n, out_n.at[pl.ds(base, b_per_w)])

  return sc_gather


_sc_gather_cache = []


def _sc_gather(pos, neg, tgt):
  if not _sc_gather_cache:
    _sc_gather_cache.append(_make_sc_gather())
  return _sc_gather_cache[0](pos, neg, tgt)


def _tc_body(x_ref, neg_ref, cp_ref, cn_ref, out_ref,
             acc_ref, xn_ref, xnb_ref):
  i = pl.program_id(0)

  @pl.when(i == 0)
  def _init():
    x = x_ref[...]
    nrm = jnp.sqrt(jnp.sum(x * x, axis=1, keepdims=True))
    xn = x / (nrm + 1e-12)
    xn_ref[...] = xn
    xnb_ref[...] = xn.astype(jnp.bfloat16)
    acc_ref[...] = jnp.zeros_like(acc_ref)

  nb = neg_ref[...].astype(jnp.bfloat16)
  s = lax.dot_general(xnb_ref[...], nb, (((1,), (1,)), ((), ())),
                      preferred_element_type=jnp.float32)
  acc_ref[...] += jnp.sum(jnp.exp(s * INV_TEMP), axis=1, keepdims=True)

  @pl.when(i == NUM_BLOCKS - 1)
  def _fini():
    xn = xn_ref[...]
    l_pos = jnp.sum(xn * cp_ref[...], axis=1, keepdims=True)
    t_neg = jnp.sum(xn * cn_ref[...], axis=1, keepdims=True)
    l_neg = acc_ref[...] - jnp.exp(t_neg * INV_TEMP) + jnp.exp(l_pos * INV_TEMP)
    per_sample = l_pos * INV_TEMP - jnp.log(l_neg)
    out_ref[...] = jnp.reshape(-jnp.mean(per_sample), (1, 1))


def _tc_loss(x, neg, cp, cn):
  return pl.pallas_call(
      _tc_body,
      grid=(NUM_BLOCKS,),
      in_specs=[
          pl.BlockSpec((B, D), lambda i: (0, 0)),
          pl.BlockSpec((ROWS_PER_BLOCK, D), lambda i: (i, 0)),
          pl.BlockSpec((B, D), lambda i: (0, 0)),
          pl.BlockSpec((B, D), lambda i: (0, 0)),
      ],
      out_specs=pl.BlockSpec((1, 1), lambda i: (0, 0)),
      out_shape=jax.ShapeDtypeStruct((1, 1), jnp.float32),
      scratch_shapes=[
          pltpu.VMEM((B, 1), jnp.float32),
          pltpu.VMEM((B, D), jnp.float32),
          pltpu.VMEM((B, D), jnp.bfloat16),
      ],
  )(x, neg, cp, cn)


def kernel(inputs, pos_protomemory, neg_protomemory, targets, indexes):
  del indexes
  cp, cn = _sc_gather(pos_protomemory, neg_protomemory,
                      targets.astype(jnp.int32))
  loss = _tc_loss(inputs, neg_protomemory, cp, cn)
  return loss[0, 0]


# speedup vs baseline: 14.9317x; 1.1427x over previous
import functools

import jax
import jax.numpy as jnp
from jax import lax
from jax.experimental import pallas as pl
from jax.experimental.pallas import tpu as pltpu
from jax.experimental.pallas import tpu_sc as plsc

B = 1024
D = 256
N = 100000
TEMP = 0.05
INV_TEMP = 1.0 / TEMP

ROWS_PER_BLOCK = 4000
NUM_BLOCKS = N // ROWS_PER_BLOCK

LOG2E = 1.4426950408889634
EXP2_SCALE = INV_TEMP * LOG2E


def _make_sc_gather():
  info = plsc.get_sparse_core_info()
  nw = info.num_cores * info.num_subcores
  b_per_w = B // nw
  mesh = plsc.VectorSubcoreMesh(core_axis_name="c", subcore_axis_name="s")

  @functools.partial(
      pl.kernel,
      mesh=mesh,
      out_type=[
          jax.ShapeDtypeStruct((B, D), jnp.float32),
          jax.ShapeDtypeStruct((B, D), jnp.float32),
      ],
      scratch_types=[
          pltpu.VMEM((b_per_w,), jnp.int32),
          pltpu.VMEM((b_per_w, D), jnp.float32),
          pltpu.VMEM((b_per_w, D), jnp.float32),
          pltpu.SemaphoreType.DMA,
          pltpu.SemaphoreType.DMA,
      ],
  )
  def sc_gather(pos_hbm, neg_hbm, tgt_hbm, out_p, out_n,
                idx_v, rows_p, rows_n, sem_p, sem_n):
    wid = lax.axis_index("s") * info.num_cores + lax.axis_index("c")
    base = wid * b_per_w
    pltpu.sync_copy(tgt_hbm.at[pl.ds(base, b_per_w)], idx_v)
    dma_p = pltpu.async_copy(pos_hbm.at[idx_v], rows_p, sem_p)
    dma_n = pltpu.async_copy(neg_hbm.at[idx_v], rows_n, sem_n)
    dma_p.wait()
    dma_n.wait()
    pltpu.sync_copy(rows_p, out_p.at[pl.ds(base, b_per_w)])
    pltpu.sync_copy(rows_n, out_n.at[pl.ds(base, b_per_w)])

  return sc_gather


_sc_gather_cache = []


def _sc_gather(pos, neg, tgt):
  if not _sc_gather_cache:
    _sc_gather_cache.append(_make_sc_gather())
  return _sc_gather_cache[0](pos, neg, tgt)


def _tc_body(x_ref, neg_ref, cp_ref, cn_ref, out_ref,
             acc_ref, xn_ref, xnb_ref):
  i = pl.program_id(0)

  @pl.when(i == 0)
  def _init():
    x = x_ref[...]
    nrm = jnp.sqrt(jnp.sum(x * x, axis=1, keepdims=True))
    xn = x / (nrm + 1e-12)
    xn_ref[...] = xn
    xnb_ref[...] = (xn * EXP2_SCALE).astype(jnp.bfloat16)
    acc_ref[...] = jnp.zeros_like(acc_ref)

  nb = neg_ref[...].astype(jnp.bfloat16)
  s = lax.dot_general(xnb_ref[...], nb, (((1,), (1,)), ((), ())),
                      preferred_element_type=jnp.float32)
  acc_ref[...] += jnp.sum(jnp.exp2(s), axis=1, keepdims=True)

  @pl.when(i == NUM_BLOCKS - 1)
  def _fini():
    xn = xn_ref[...]
    l_pos = jnp.sum(xn * cp_ref[...], axis=1, keepdims=True)
    t_neg = jnp.sum(xn * cn_ref[...], axis=1, keepdims=True)
    l_neg = acc_ref[...] - jnp.exp(t_neg * INV_TEMP) + jnp.exp(l_pos * INV_TEMP)
    per_sample = l_pos * INV_TEMP - jnp.log(l_neg)
    out_ref[...] = jnp.reshape(-jnp.mean(per_sample), (1, 1))


def _tc_loss(x, neg, cp, cn):
  return pl.pallas_call(
      _tc_body,
      grid=(NUM_BLOCKS,),
      in_specs=[
          pl.BlockSpec((B, D), lambda i: (0, 0)),
          pl.BlockSpec((ROWS_PER_BLOCK, D), lambda i: (i, 0)),
          pl.BlockSpec((B, D), lambda i: (0, 0)),
          pl.BlockSpec((B, D), lambda i: (0, 0)),
      ],
      out_specs=pl.BlockSpec((1, 1), lambda i: (0, 0)),
      out_shape=jax.ShapeDtypeStruct((1, 1), jnp.float32),
      scratch_shapes=[
          pltpu.VMEM((B, 1), jnp.float32),
          pltpu.VMEM((B, D), jnp.float32),
          pltpu.VMEM((B, D), jnp.bfloat16),
      ],
  )(x, neg, cp, cn)


def kernel(inputs, pos_protomemory, neg_protomemory, targets, indexes):
  del indexes
  cp, cn = _sc_gather(pos_protomemory, neg_protomemory,
                      targets.astype(jnp.int32))
  loss = _tc_loss(inputs, neg_protomemory, cp, cn)
  return loss[0, 0]
